# initial kernel scaffold (unmeasured)
import jax
import jax.numpy as jnp
from jax import lax
from jax.experimental import pallas as pl
from jax.experimental.pallas import tpu as pltpu

N_DEV = 16


def kernel(x, w_mat):
    K, kper = x.shape
    _, N = w_mat.shape
    m_per = K // N_DEV

    def body(x_ref, w_ref, out_ref, xsend, xbuf, send_sems, recv_sems):
        my = lax.axis_index("i")

        xsend[:, :] = x_ref[:, :].astype(jnp.bfloat16)

        barrier_sem = pltpu.get_barrier_semaphore()
        for s in range(1, N_DEV):
            peer = lax.rem(my + s, N_DEV)
            pl.semaphore_signal(
                barrier_sem, inc=1,
                device_id=(peer,), device_id_type=pl.DeviceIdType.MESH,
            )
        pl.semaphore_wait(barrier_sem, N_DEV - 1)

        sends = []
        for s in range(1, N_DEV):
            j = lax.rem(my + s, N_DEV)
            rdma = pltpu.make_async_remote_copy(
                src_ref=xsend.at[pl.ds(j * m_per, m_per), :],
                dst_ref=xbuf.at[s],
                send_sem=send_sems.at[s],
                recv_sem=recv_sems.at[s],
                device_id=(j,),
                device_id_type=pl.DeviceIdType.MESH,
            )
            rdma.start()
            sends.append(rdma)

        def w_chunk(u):
            return w_ref[pl.ds(u * kper, kper), :].astype(jnp.bfloat16)

        acc = jnp.dot(
            xsend[pl.ds(my * m_per, m_per), :], w_chunk(my),
            preferred_element_type=jnp.float32,
        )

        for s in range(1, N_DEV):
            u = lax.rem(my - s + N_DEV, N_DEV)
            recv = pltpu.make_async_remote_copy(
                src_ref=xsend.at[pl.ds(0, m_per), :],
                dst_ref=xbuf.at[s],
                send_sem=send_sems.at[s],
                recv_sem=recv_sems.at[s],
                device_id=(my,),
                device_id_type=pl.DeviceIdType.MESH,
            )
            recv.wait_recv()
            acc = acc + jnp.dot(
                xbuf[s], w_chunk(u), preferred_element_type=jnp.float32,
            )

        out_ref[:, :] = jnp.maximum(acc, 0.0)

        for rdma in sends:
            rdma.wait_send()

    return pl.pallas_call(
        body,
        out_shape=jax.ShapeDtypeStruct((m_per, N), jnp.float32),
        in_specs=[
            pl.BlockSpec(memory_space=pltpu.VMEM),
            pl.BlockSpec(memory_space=pltpu.VMEM),
        ],
        out_specs=pl.BlockSpec(memory_space=pltpu.VMEM),
        scratch_shapes=[
            pltpu.VMEM((K, kper), jnp.bfloat16),
            pltpu.VMEM((N_DEV, m_per, kper), jnp.bfloat16),
            pltpu.SemaphoreType.DMA((N_DEV,)),
            pltpu.SemaphoreType.DMA((N_DEV,)),
        ],
        compiler_params=pltpu.CompilerParams(collective_id=0),
    )(x, w_mat)


# baseline (device time: 43913 ns/iter reference)
import jax
import jax.numpy as jnp
from jax import lax
from jax.experimental import pallas as pl
from jax.experimental.pallas import tpu as pltpu

N_DEV = 16


def kernel(x, w_mat):
    K, kper = x.shape
    _, N = w_mat.shape
    m_per = K // N_DEV

    def body(x_ref, w_ref, out_ref, xsend, xbuf, send_sems, recv_sems):
        my = lax.axis_index("i")

        xsend[:, :] = x_ref[:, :].astype(jnp.bfloat16)

        barrier_sem = pltpu.get_barrier_semaphore()
        for s in range(1, N_DEV):
            peer = lax.rem(my + s, N_DEV)
            pl.semaphore_signal(
                barrier_sem, inc=1,
                device_id=(peer,), device_id_type=pl.DeviceIdType.MESH,
            )
        pl.semaphore_wait(barrier_sem, N_DEV - 1)

        sends = []
        for s in range(1, N_DEV):
            j = lax.rem(my + s, N_DEV)
            rdma = pltpu.make_async_remote_copy(
                src_ref=xsend.at[pl.ds(j * m_per, m_per), :],
                dst_ref=xbuf.at[s],
                send_sem=send_sems.at[s],
                recv_sem=recv_sems.at[s],
                device_id=(j,),
                device_id_type=pl.DeviceIdType.MESH,
            )
            rdma.start()
            sends.append(rdma)

        def w_chunk(u):
            return w_ref[pl.ds(u * kper, kper), :].astype(jnp.bfloat16)

        acc = jnp.dot(
            xsend[pl.ds(my * m_per, m_per), :], w_chunk(my),
            preferred_element_type=jnp.float32,
        )

        for s in range(1, N_DEV):
            u = lax.rem(my - s + N_DEV, N_DEV)
            recv = pltpu.make_async_remote_copy(
                src_ref=xsend.at[pl.ds(0, m_per), :],
                dst_ref=xbuf.at[s],
                send_sem=send_sems.at[s],
                recv_sem=recv_sems.at[s],
                device_id=(my,),
                device_id_type=pl.DeviceIdType.MESH,
            )
            recv.wait_recv()
            acc = acc + jnp.dot(
                xbuf[s], w_chunk(u), preferred_element_type=jnp.float32,
            )

        out_ref[:, :] = jnp.maximum(acc, 0.0)

        for rdma in sends:
            rdma.wait_send()

    return pl.pallas_call(
        body,
        out_shape=jax.ShapeDtypeStruct((m_per, N), jnp.float32),
        in_specs=[
            pl.BlockSpec(memory_space=pltpu.VMEM),
            pl.BlockSpec(memory_space=pltpu.VMEM),
        ],
        out_specs=pl.BlockSpec(memory_space=pltpu.VMEM),
        scratch_shapes=[
            pltpu.VMEM((K, kper), jnp.bfloat16),
            pltpu.VMEM((N_DEV, m_per, kper), jnp.bfloat16),
            pltpu.SemaphoreType.DMA((N_DEV,)),
            pltpu.SemaphoreType.DMA((N_DEV,)),
        ],
        compiler_params=pltpu.CompilerParams(
            collective_id=0,
            vmem_limit_bytes=60 * 1024 * 1024,
        ),
    )(x, w_mat)


# device time: 39092 ns/iter; 1.1233x vs baseline; 1.1233x over previous
import jax
import jax.numpy as jnp
from jax import lax
from jax.experimental import pallas as pl
from jax.experimental.pallas import tpu as pltpu

N_DEV = 16


def kernel(x, w_mat):
    K, kper = x.shape
    _, N = w_mat.shape
    m_per = K // N_DEV

    def body(x_ref, w_hbm, out_ref, xsend, xbuf, wbuf,
             send_sems, recv_sems, wsems):
        my = lax.axis_index("i")

        def w_dma(u, slot):
            return pltpu.make_async_copy(
                w_hbm.at[pl.ds(u * kper, kper), :],
                wbuf.at[slot],
                wsems.at[slot],
            )

        w_dma(my, 0).start()

        xsend[:, :] = x_ref[:, :].astype(jnp.bfloat16)

        barrier_sem = pltpu.get_barrier_semaphore()
        for s in range(1, N_DEV):
            peer = lax.rem(my + s, N_DEV)
            pl.semaphore_signal(
                barrier_sem, inc=1,
                device_id=(peer,), device_id_type=pl.DeviceIdType.MESH,
            )
        pl.semaphore_wait(barrier_sem, N_DEV - 1)

        sends = []
        for s in range(1, N_DEV):
            j = lax.rem(my + s, N_DEV)
            rdma = pltpu.make_async_remote_copy(
                src_ref=xsend.at[pl.ds(j * m_per, m_per), :],
                dst_ref=xbuf.at[s],
                send_sem=send_sems.at[s],
                recv_sem=recv_sems.at[s],
                device_id=(j,),
                device_id_type=pl.DeviceIdType.MESH,
            )
            rdma.start()
            sends.append(rdma)

        acc = jnp.zeros((m_per, N), jnp.float32)
        for s in range(N_DEV):
            u = lax.rem(my - s + N_DEV, N_DEV)
            slot = s % 2
            if s + 1 < N_DEV:
                u_next = lax.rem(my - s - 1 + N_DEV, N_DEV)
                w_dma(u_next, (s + 1) % 2).start()
            w_dma(u, slot).wait()
            if s == 0:
                xchunk = xsend[pl.ds(my * m_per, m_per), :]
            else:
                recv = pltpu.make_async_remote_copy(
                    src_ref=xsend.at[pl.ds(0, m_per), :],
                    dst_ref=xbuf.at[s],
                    send_sem=send_sems.at[s],
                    recv_sem=recv_sems.at[s],
                    device_id=(my,),
                    device_id_type=pl.DeviceIdType.MESH,
                )
                recv.wait_recv()
                xchunk = xbuf[s]
            acc = acc + jnp.dot(
                xchunk, wbuf[slot].astype(jnp.bfloat16),
                preferred_element_type=jnp.float32,
            )

        out_ref[:, :] = jnp.maximum(acc, 0.0)

        for rdma in sends:
            rdma.wait_send()

    return pl.pallas_call(
        body,
        out_shape=jax.ShapeDtypeStruct((m_per, N), jnp.float32),
        in_specs=[
            pl.BlockSpec(memory_space=pltpu.VMEM),
            pl.BlockSpec(memory_space=pl.ANY),
        ],
        out_specs=pl.BlockSpec(memory_space=pltpu.VMEM),
        scratch_shapes=[
            pltpu.VMEM((K, kper), jnp.bfloat16),
            pltpu.VMEM((N_DEV, m_per, kper), jnp.bfloat16),
            pltpu.VMEM((2, kper, N), jnp.float32),
            pltpu.SemaphoreType.DMA((N_DEV,)),
            pltpu.SemaphoreType.DMA((N_DEV,)),
            pltpu.SemaphoreType.DMA((2,)),
        ],
        compiler_params=pltpu.CompilerParams(
            collective_id=0,
            vmem_limit_bytes=60 * 1024 * 1024,
        ),
    )(x, w_mat)
